# baseline (speedup 1.0000x reference)
import jax
import jax.numpy as jnp
from jax import lax
from jax.experimental import pallas as pl
from jax.experimental.pallas import tpu as pltpu, tpu_sc as plsc

_D = 2048
_CH = 512
_NBUF = 4
_NB = 15
_NSTEP = _NB * 4

_TSPLIT = 7680
_SC_ROWS_PER_W = 64
_SC_CHW = 16384
_SC_NCHUNK = _SC_ROWS_PER_W * _D // _SC_CHW


def _tc_base(s):
    i = s // 4
    b = s % 4
    return b * 8192 + i * _CH, i


def _x_copy(x_hbm, xbuf, xsem, s):
    base, _ = _tc_base(s)
    return pltpu.make_async_copy(
        x_hbm.at[pl.ds(base, _CH), :], xbuf.at[s % _NBUF], xsem.at[s % _NBUF])


def _o_copy(o_hbm, obuf, osem, s):
    base, _ = _tc_base(s)
    return pltpu.make_async_copy(
        obuf.at[s % _NBUF], o_hbm.at[pl.ds(base, _CH), :], osem.at[s % _NBUF])


def _e_copy(e_hbm, ebuf, esem, i):
    return pltpu.make_async_copy(
        e_hbm.at[pl.ds(i * _CH, _CH), :], ebuf.at[i % 2], esem.at[i % 2])


def _tc_body(x_hbm, e_hbm, o_hbm, xbuf, ebuf, obuf, xsem, esem, osem):
    @pl.when(pl.program_id(0) == 0)
    def _prologue():
        for c in range(_NBUF):
            _x_copy(x_hbm, xbuf, xsem, c).start()
        _e_copy(e_hbm, ebuf, esem, 0).start()
        _e_copy(e_hbm, ebuf, esem, 1).start()

    s = pl.program_id(0)
    _, i = _tc_base(s)
    b = s % 4

    @pl.when(b == 0)
    def _embed_turnover():
        @pl.when(jnp.logical_and(i >= 1, i < _NB - 1))
        def _prefetch_next():
            _e_copy(e_hbm, ebuf, esem, i + 1).start()

        _e_copy(e_hbm, ebuf, esem, i).wait()

    _x_copy(x_hbm, xbuf, xsem, s).wait()

    @pl.when(s >= _NBUF)
    def _drain_out():
        _o_copy(o_hbm, obuf, osem, s - _NBUF).wait()

    obuf[s % _NBUF] = xbuf[s % _NBUF] + ebuf[i % 2]
    _o_copy(o_hbm, obuf, osem, s).start()

    @pl.when(s < _NSTEP - _NBUF)
    def _refill_x():
        _x_copy(x_hbm, xbuf, xsem, s + _NBUF).start()

    @pl.when(s == _NSTEP - 1)
    def _epilogue():
        for k in range(_NBUF):
            _o_copy(o_hbm, obuf, osem, s - (_NBUF - 1) + k).wait()


def _sc_body(x_hbm, e_hbm, o_hbm):
    def scoped(xb, eb, ob, xsem, esem, osem):
        wid = lax.axis_index("s") * 2 + lax.axis_index("c")
        b = wid // 8
        sub = wid % 8
        xbase = (b * 8192 + _TSPLIT + sub * _SC_ROWS_PER_W) * _D
        obase = (b * (8192 - _TSPLIT) + sub * _SC_ROWS_PER_W) * _D
        ebase = (_TSPLIT + sub * _SC_ROWS_PER_W) * _D

        def x_copy(c, slot):
            return pltpu.make_async_copy(
                x_hbm.at[pl.ds(xbase + c * _SC_CHW, _SC_CHW)],
                xb.at[slot], xsem.at[slot])

        def e_copy(c, slot):
            return pltpu.make_async_copy(
                e_hbm.at[pl.ds(ebase + c * _SC_CHW, _SC_CHW)],
                eb.at[slot], esem.at[slot])

        def o_copy(c, slot):
            return pltpu.make_async_copy(
                ob.at[slot], o_hbm.at[pl.ds(obase + c * _SC_CHW, _SC_CHW)],
                osem.at[slot])

        x_copy(0, 0).start()
        e_copy(0, 0).start()
        x_copy(1, 1).start()
        e_copy(1, 1).start()

        def step(c, carry):
            slot = lax.rem(c, 2)

            @pl.when(c >= 2)
            def _drain():
                o_copy(c - 2, slot).wait()

            x_copy(c, slot).wait()
            e_copy(c, slot).wait()

            def add16(j, c2):
                ob[slot, pl.ds(j * 16, 16)] = (
                    xb[slot, pl.ds(j * 16, 16)] + eb[slot, pl.ds(j * 16, 16)])
                return c2

            lax.fori_loop(0, _SC_CHW // 16, add16, 0, unroll=8)
            o_copy(c, slot).start()

            @pl.when(c + 2 < _SC_NCHUNK)
            def _refill():
                x_copy(c + 2, slot).start()
                e_copy(c + 2, slot).start()

            return carry

        lax.fori_loop(0, _SC_NCHUNK, step, 0)
        o_copy(_SC_NCHUNK - 2, lax.rem(_SC_NCHUNK - 2, 2)).wait()
        o_copy(_SC_NCHUNK - 1, lax.rem(_SC_NCHUNK - 1, 2)).wait()

    pl.run_scoped(
        scoped,
        pltpu.VMEM((2, _SC_CHW), jnp.float32),
        pltpu.VMEM((2, _SC_CHW), jnp.float32),
        pltpu.VMEM((2, _SC_CHW), jnp.float32),
        pltpu.SemaphoreType.DMA((2,)),
        pltpu.SemaphoreType.DMA((2,)),
        pltpu.SemaphoreType.DMA((2,)),
    )


def kernel(x, embed):
    B, T, D = x.shape
    x2 = x.reshape(B * T, D)
    x1 = x.reshape(B * T * D)
    e1 = embed.reshape(T * D)

    out_sc = pl.kernel(
        _sc_body,
        out_type=jax.ShapeDtypeStruct((B * (T - _TSPLIT) * D,), jnp.float32),
        mesh=plsc.VectorSubcoreMesh(core_axis_name="c", subcore_axis_name="s"),
    )(x1, e1)

    out_tc = pl.pallas_call(
        _tc_body,
        grid=(_NSTEP,),
        in_specs=[
            pl.BlockSpec(memory_space=pltpu.HBM),
            pl.BlockSpec(memory_space=pltpu.HBM),
        ],
        out_specs=pl.BlockSpec(memory_space=pltpu.HBM),
        out_shape=jax.ShapeDtypeStruct((B * T, D), x.dtype),
        scratch_shapes=[
            pltpu.VMEM((_NBUF, _CH, _D), jnp.float32),
            pltpu.VMEM((2, _CH, _D), jnp.float32),
            pltpu.VMEM((_NBUF, _CH, _D), jnp.float32),
            pltpu.SemaphoreType.DMA((_NBUF,)),
            pltpu.SemaphoreType.DMA((2,)),
            pltpu.SemaphoreType.DMA((_NBUF,)),
        ],
        compiler_params=pltpu.CompilerParams(
            dimension_semantics=("arbitrary",)),
    )(x2, embed)

    out3 = out_tc.reshape(B, T, D)
    sc3 = out_sc.reshape(B, T - _TSPLIT, D)
    return lax.dynamic_update_slice(out3, sc3, (0, _TSPLIT, 0))
